# baseline (speedup 1.0000x reference)
import jax
import jax.numpy as jnp
from jax import lax
from jax.experimental import pallas as pl
from jax.experimental.pallas import tpu as pltpu
from jax.experimental.pallas import tpu_sc as plsc

B = 16384
D = 64
NC = 2
NS = 16
L = 16
NW = NC * NS
BPW = B // NW

NU = 1000000
NI = 100000
UCW = 512
ICW = 128
UTAIL = (NU // ICW) * ICW
ITAIL = (NI // ICW) * ICW
NUCH = UTAIL // UCW
NICH = ITAIL // ICW
KU = -(-NUCH // NW)
KI = -(-NICH // NW)
LCAP = 4096
ACAP = 1024
BIG = 1 << 30


def _extract_body(users_hbm, items_hbm, utab_hbm, itab_hbm, u_hbm, v_hbm,
                  au, ai, ulist, ubl, ilist, ibl, acol, ab,
                  uchunk, ichunk, rbuf, bsbuf, cnts, semw):
    wid = lax.axis_index("s") * NC + lax.axis_index("c")
    iota16 = lax.iota(jnp.int32, L)

    for kk in range(8):
        bsbuf[0, pl.ds(kk * L, L)] = jnp.full((L,), B, jnp.int32)
    pltpu.sync_copy(users_hbm, au)
    pltpu.sync_copy(items_hbm, ai)

    cnts[0] = 0
    cnts[1] = 0

    def build(t, carry):
        b = t * L + iota16
        u = au[pl.ds(t * L, L)]
        mu = (((u >> 9) & 31) == wid) & (u < UTAIL)
        cu = cnts[0]
        plsc.store_compressed(ulist.at[pl.ds(cu, L)], u, mask=mu)
        plsc.store_compressed(ubl.at[pl.ds(cu, L)], b, mask=mu)
        cnts[0] = cu + plsc.all_reduce_population_count(mu)[0]
        i = ai[pl.ds(t * L, L)]
        mi = (((i >> 7) & 31) == wid) & (i < ITAIL)
        ci = cnts[1]
        plsc.store_compressed(ilist.at[pl.ds(ci, L)], i, mask=mi)
        plsc.store_compressed(ibl.at[pl.ds(ci, L)], b, mask=mi)
        cnts[1] = ci + plsc.all_reduce_population_count(mi)[0]
        return carry

    lax.fori_loop(0, B // L, build, 0)
    ulist[pl.ds(cnts[0], L)] = jnp.full((L,), BIG, jnp.int32)
    ilist[pl.ds(cnts[1], L)] = jnp.full((L,), BIG, jnp.int32)

    def stream(tab_hbm, lst, bls, cnt, chunkbuf, cw_log2, k_iters, n_chunks,
               out_hbm):
        cw = 1 << cw_log2
        colmask = cw - 1
        nvreg = (cnt + L - 1) >> 4

        def per_chunk(k, carry):
            cid = wid + NW * k

            @pl.when(cid < n_chunks)
            def _():
                off = pl.multiple_of(cid * cw, cw)
                pltpu.sync_copy(tab_hbm.at[:, pl.ds(off, cw)], chunkbuf)
                cnts[2] = 0

                def scan(t, c2):
                    lv = lst[pl.ds(t * L, L)]
                    bv = bls[pl.ds(t * L, L)]
                    m = (lv >> cw_log2) == cid
                    plsc.store_compressed(acol.at[pl.ds(c2, L)],
                                          lv & colmask, mask=m)
                    plsc.store_compressed(ab.at[pl.ds(c2, L)], bv, mask=m)
                    return c2 + plsc.all_reduce_population_count(m)[0]

                c2 = lax.fori_loop(0, nvreg, scan, 0)
                ngroup = (c2 + L - 1) >> 4

                def group(g, carry2):
                    col = acol[pl.ds(g * L, L)] & colmask
                    bs = ab[pl.ds(g * L, L)]
                    gm = iota16 < (c2 - g * L)
                    bs = jnp.where(gm, bs, B)
                    for d in range(D):
                        dd = jnp.full((L,), d, jnp.int32)
                        vals = plsc.load_gather(chunkbuf, [dd, col])
                        plsc.store_scatter(rbuf, [iota16, dd], vals)
                    bsbuf[0, pl.ds(0, L)] = bs
                    pltpu.async_copy(rbuf, out_hbm.at[bsbuf.at[0]], semw)
                    cnts[3] = cnts[3] + 1
                    return carry2

                lax.fori_loop(0, ngroup, group, 0)

            return carry

        lax.fori_loop(0, k_iters, per_chunk, 0)

    cnts[3] = 0
    stream(utab_hbm, ulist, ubl, cnts[0], uchunk, 9, KU, NUCH, u_hbm)
    stream(itab_hbm, ilist, ibl, cnts[1], ichunk, 7, KI, NICH, v_hbm)

    def drain(t, carry):
        pltpu.make_async_copy(
            rbuf, u_hbm.at[bsbuf.at[0]], semw).wait()
        return carry

    lax.fori_loop(0, cnts[3], drain, 0)


def _dot_body(users_hbm, items_hbm, u_hbm, v_hbm, utail_hbm, itail_hbm,
              out_hbm, uidx, iidx, ubuf, vbuf, utb, itb, outv):
    wid = lax.axis_index("s") * NC + lax.axis_index("c")
    base = wid * BPW
    iota16 = lax.iota(jnp.int32, L)

    pltpu.sync_copy(users_hbm.at[pl.ds(base, BPW)], uidx)
    pltpu.sync_copy(items_hbm.at[pl.ds(base, BPW)], iidx)
    pltpu.sync_copy(utail_hbm, utb)
    pltpu.sync_copy(itail_hbm, itb)

    def subchunk(s, carry):
        pltpu.sync_copy(u_hbm.at[pl.ds(base + s * 128, 128)], ubuf)
        pltpu.sync_copy(v_hbm.at[pl.ds(base + s * 128, 128)], vbuf)

        def group(g, carry2):
            lanes = g * L + iota16
            uvec = uidx[pl.ds(s * 128 + g * L, L)]
            ivec = iidx[pl.ds(s * 128 + g * L, L)]
            mu = uvec >= UTAIL
            mi = ivec >= ITAIL
            tuc = jnp.maximum(uvec - UTAIL, 0)
            tic = jnp.maximum(ivec - ITAIL, 0)
            acc = jnp.zeros((L,), jnp.float32)
            for d in range(D):
                dd = jnp.full((L,), d, jnp.int32)
                uv = plsc.load_gather(ubuf, [lanes, dd])
                tv = plsc.load_gather(utb, [tuc, dd], mask=mu)
                uv = jnp.where(mu, tv, uv)
                vv = plsc.load_gather(vbuf, [lanes, dd])
                tw = plsc.load_gather(itb, [tic, dd], mask=mi)
                vv = jnp.where(mi, tw, vv)
                acc = acc + uv * vv
            outv[pl.ds(s * 128 + g * L, L)] = acc
            return carry2

        lax.fori_loop(0, 128 // L, group, 0)
        return carry

    lax.fori_loop(0, BPW // 128, subchunk, 0)

    pltpu.sync_copy(outv, out_hbm.at[pl.ds(base, BPW)])


def kernel(users, items, users_table, items_table):
    ut_t = users_table.T
    it_t = items_table.T
    utail = users_table[UTAIL:]
    itail = items_table[ITAIL:]
    users = users.astype(jnp.int32)
    items = items.astype(jnp.int32)
    mesh = plsc.VectorSubcoreMesh(core_axis_name="c", subcore_axis_name="s")

    extract = pl.kernel(
        _extract_body,
        out_type=(jax.ShapeDtypeStruct((B + 8, 2 * D), jnp.float32),
                  jax.ShapeDtypeStruct((B + 8, 2 * D), jnp.float32)),
        mesh=mesh,
        compiler_params=pltpu.CompilerParams(needs_layout_passes=False),
        scratch_types=[
            pltpu.VMEM((B,), jnp.int32),
            pltpu.VMEM((B,), jnp.int32),
            pltpu.VMEM((LCAP + L,), jnp.int32),
            pltpu.VMEM((LCAP + L,), jnp.int32),
            pltpu.VMEM((LCAP + L,), jnp.int32),
            pltpu.VMEM((LCAP + L,), jnp.int32),
            pltpu.VMEM((ACAP + L,), jnp.int32),
            pltpu.VMEM((ACAP + L,), jnp.int32),
            pltpu.VMEM((D, UCW), jnp.float32),
            pltpu.VMEM((D, ICW), jnp.float32),
            pltpu.VMEM((128, 2 * D), jnp.float32),
            pltpu.VMEM((1, 128), jnp.int32),
            pltpu.SMEM((4,), jnp.int32),
            pltpu.SemaphoreType.DMA,
        ],
    )
    u_rows, v_rows = extract(users, items, ut_t, it_t)

    dot = pl.kernel(
        _dot_body,
        out_type=jax.ShapeDtypeStruct((B,), jnp.float32),
        mesh=mesh,
        compiler_params=pltpu.CompilerParams(needs_layout_passes=False),
        scratch_types=[
            pltpu.VMEM((BPW,), jnp.int32),
            pltpu.VMEM((BPW,), jnp.int32),
            pltpu.VMEM((128, 2 * D), jnp.float32),
            pltpu.VMEM((128, 2 * D), jnp.float32),
            pltpu.VMEM((D, D), jnp.float32),
            pltpu.VMEM((D // 2, D), jnp.float32),
            pltpu.VMEM((BPW,), jnp.float32),
        ],
    )
    return dot(users, items, u_rows, v_rows, utail, itail)


# speedup vs baseline: 57.2345x; 57.2345x over previous
import jax
import jax.numpy as jnp
from jax import lax
from jax.experimental import pallas as pl
from jax.experimental.pallas import tpu as pltpu
from jax.experimental.pallas import tpu_sc as plsc

B = 16384
D = 64
NC = 2
NS = 16
L = 16
NW = NC * NS
BPW = B // NW

NU = 1000000
NI = 100000
RB = 8
UCW = 512
UTAIL = (NU // 128) * 128
NUCH = UTAIL // UCW
KU = -(-NUCH // NW)
LCAP = 1024
ACAP = 1024
BIG = 1 << 30


def _main_body(users_hbm, items_hbm, utab_hbm, itab_hbm, part_hbm,
               ubuild, ai, ulist, ubl, acol, ab, uchunk0, uchunk1,
               vblock, scores, cnts, semr0, semr1, semv):
    wid = lax.axis_index("s") * NC + lax.axis_index("c")
    iota16 = lax.iota(jnp.int32, L)

    pltpu.sync_copy(items_hbm, ai)

    def zero(t, carry):
        scores[pl.ds(t * L, L)] = jnp.zeros((L,), jnp.float32)
        return carry

    lax.fori_loop(0, (B + L) // L, zero, 0)

    cnts[0] = 0

    def build_outer(p, carry):
        pltpu.sync_copy(users_hbm.at[pl.ds(p * BPW, BPW)], ubuild)

        def build(t, carry2):
            b = p * BPW + t * L + iota16
            u = ubuild[pl.ds(t * L, L)]
            mu = (((u >> 9) & 31) == wid) & (u < UTAIL)
            cu = cnts[0]
            plsc.store_compressed(ulist.at[pl.ds(cu, L)], u, mask=mu)
            plsc.store_compressed(ubl.at[pl.ds(cu, L)], b, mask=mu)
            cnts[0] = cu + plsc.all_reduce_population_count(mu)[0]
            return carry2

        lax.fori_loop(0, BPW // L, build, 0)
        return carry

    lax.fori_loop(0, B // BPW, build_outer, 0)
    ulist[pl.ds(cnts[0], L)] = jnp.full((L,), BIG, jnp.int32)
    cnt = cnts[0]
    nvreg = (cnt + L - 1) >> 4

    def issue(k, chunkbuf, semr):
        cid = wid + NW * k

        @pl.when(cid < NUCH)
        def _():
            off = pl.multiple_of(cid * UCW, UCW)
            pltpu.async_copy(utab_hbm.at[:, pl.ds(off, UCW)], chunkbuf, semr)

    def process(k, chunkbuf, semr):
        cid = wid + NW * k

        @pl.when(cid < NUCH)
        def _():
            pltpu.make_async_copy(
                utab_hbm.at[:, pl.ds(0, UCW)], chunkbuf, semr).wait()

            def scan(t, c2):
                lv = ulist[pl.ds(t * L, L)]
                bv = ubl[pl.ds(t * L, L)]
                m = (lv >> 9) == cid
                plsc.store_compressed(acol.at[pl.ds(c2, L)],
                                      lv & (UCW - 1), mask=m)
                plsc.store_compressed(ab.at[pl.ds(c2, L)], bv, mask=m)
                return c2 + plsc.all_reduce_population_count(m)[0]

            c2 = lax.fori_loop(0, nvreg, scan, 0)
            ngroup = (c2 + L - 1) >> 4

            def group(g, carry2):
                col = acol[pl.ds(g * L, L)] & (UCW - 1)
                bs = ab[pl.ds(g * L, L)] & (B - 1)
                gm = iota16 < (c2 - g * L)
                iv = plsc.load_gather(ai, [bs])
                ivb = iv >> 3
                ivs = iv & 7
                for j in range(L):
                    pltpu.async_copy(
                        itab_hbm.at[ivb[j]], vblock.at[j], semv)
                pltpu.make_async_copy(
                    itab_hbm.at[pl.ds(0, L)], vblock, semv).wait()
                acc = jnp.zeros((L,), jnp.float32)
                for d in range(D):
                    dd = jnp.full((L,), d, jnp.int32)
                    u = plsc.load_gather(chunkbuf, [dd, col])
                    v = plsc.load_gather(vblock, [iota16, ivs, dd])
                    acc = acc + u * v
                bs_dump = jnp.where(gm, bs, B)
                plsc.store_scatter(scores, [bs_dump], acc)
                return carry2

            lax.fori_loop(0, ngroup, group, 0)

    issue(0, uchunk0, semr0)

    def step(t, carry):
        issue(2 * t + 1, uchunk1, semr1)
        process(2 * t, uchunk0, semr0)

        @pl.when(t + 1 < KU // 2)
        def _():
            issue(2 * t + 2, uchunk0, semr0)

        process(2 * t + 1, uchunk1, semr1)
        return carry

    lax.fori_loop(0, KU // 2, step, 0)

    pltpu.sync_copy(scores.at[pl.ds(0, B)], part_hbm.at[pl.ds(wid * B, B)])


def _reduce_body(users_hbm, items_hbm, part_hbm, utail_hbm, itab_hbm,
                 out_hbm, uidx, iidx, pbuf, utb, vtb, accv, semp, semv):
    wid = lax.axis_index("s") * NC + lax.axis_index("c")
    base = wid * BPW
    iota16 = lax.iota(jnp.int32, L)

    pltpu.sync_copy(users_hbm.at[pl.ds(base, BPW)], uidx)
    pltpu.sync_copy(items_hbm.at[pl.ds(base, BPW)], iidx)
    pltpu.sync_copy(utail_hbm, utb)

    for w2 in range(NW):
        pltpu.async_copy(
            part_hbm.at[pl.ds(w2 * B + base, BPW)], pbuf.at[w2], semp)
    for w2 in range(NW):
        pltpu.make_async_copy(
            part_hbm.at[pl.ds(0, BPW)], pbuf.at[w2], semp).wait()

    def sum_group(g, carry):
        acc = jnp.zeros((L,), jnp.float32)
        for w2 in range(NW):
            acc = acc + pbuf[w2, pl.ds(g * L, L)]
        accv[pl.ds(g * L, L)] = acc
        return carry

    lax.fori_loop(0, BPW // L, sum_group, 0)

    def tail_group(g, carry):
        uvec = uidx[pl.ds(g * L, L)]
        mu = uvec >= UTAIL

        @pl.when(plsc.all_reduce_population_count(mu)[0] > 0)
        def _():
            ivec = iidx[pl.ds(g * L, L)]
            ivb = ivec >> 3
            ivs = ivec & 7
            tuc = jnp.maximum(uvec - UTAIL, 0)
            for j in range(L):
                pltpu.async_copy(itab_hbm.at[ivb[j]], vtb.at[j], semv)
            pltpu.make_async_copy(
                itab_hbm.at[pl.ds(0, L)], vtb, semv).wait()
            acc = jnp.zeros((L,), jnp.float32)
            for d in range(D):
                dd = jnp.full((L,), d, jnp.int32)
                ut = plsc.load_gather(utb, [tuc, dd], mask=mu)
                vt = plsc.load_gather(vtb, [iota16, ivs, dd], mask=mu)
                acc = acc + ut * vt
            old = accv[pl.ds(g * L, L)]
            accv[pl.ds(g * L, L)] = jnp.where(mu, acc, old)

        return carry

    lax.fori_loop(0, BPW // L, tail_group, 0)

    pltpu.sync_copy(accv, out_hbm.at[pl.ds(base, BPW)])


def kernel(users, items, users_table, items_table):
    ut_t = users_table.T
    it3 = items_table.reshape(NI // RB, RB, D)
    utail = users_table[UTAIL:]
    users = users.astype(jnp.int32)
    items = items.astype(jnp.int32)
    mesh = plsc.VectorSubcoreMesh(core_axis_name="c", subcore_axis_name="s")

    main = pl.kernel(
        _main_body,
        out_type=jax.ShapeDtypeStruct((NW * B,), jnp.float32),
        mesh=mesh,
        compiler_params=pltpu.CompilerParams(needs_layout_passes=False),
        scratch_types=[
            pltpu.VMEM((BPW,), jnp.int32),
            pltpu.VMEM((B,), jnp.int32),
            pltpu.VMEM((LCAP + L,), jnp.int32),
            pltpu.VMEM((LCAP + L,), jnp.int32),
            pltpu.VMEM((ACAP + L,), jnp.int32),
            pltpu.VMEM((ACAP + L,), jnp.int32),
            pltpu.VMEM((D, UCW), jnp.float32),
            pltpu.VMEM((D, UCW), jnp.float32),
            pltpu.VMEM((L, RB, D), jnp.float32),
            pltpu.VMEM((B + L,), jnp.float32),
            pltpu.SMEM((4,), jnp.int32),
            pltpu.SemaphoreType.DMA,
            pltpu.SemaphoreType.DMA,
            pltpu.SemaphoreType.DMA,
        ],
    )
    partials = main(users, items, ut_t, it3)

    reduce = pl.kernel(
        _reduce_body,
        out_type=jax.ShapeDtypeStruct((B,), jnp.float32),
        mesh=mesh,
        compiler_params=pltpu.CompilerParams(needs_layout_passes=False),
        scratch_types=[
            pltpu.VMEM((BPW,), jnp.int32),
            pltpu.VMEM((BPW,), jnp.int32),
            pltpu.VMEM((NW, BPW), jnp.float32),
            pltpu.VMEM((D, D), jnp.float32),
            pltpu.VMEM((L, RB, D), jnp.float32),
            pltpu.VMEM((BPW,), jnp.float32),
            pltpu.SemaphoreType.DMA,
            pltpu.SemaphoreType.DMA,
        ],
    )
    return reduce(users, items, partials, utail, it3)


# speedup vs baseline: 63.1235x; 1.1029x over previous
import jax
import jax.numpy as jnp
from jax import lax
from jax.experimental import pallas as pl
from jax.experimental.pallas import tpu as pltpu
from jax.experimental.pallas import tpu_sc as plsc

B = 16384
D = 64
NC = 2
NS = 16
L = 16
NW = NC * NS
BPW = B // NW

NU = 1000000
NI = 100000
RB = 8
UCW = 512
UTAIL = (NU // 128) * 128
NUCH = UTAIL // UCW
KU = -(-NUCH // NW)
LCAP = 1024
ACAP = 1024
BIG = 1 << 30


def _main_body(users_hbm, items_hbm, utab_hbm, itab_hbm, part_hbm,
               ubuild, ai, ulist, ubl, acol, ab, uchunk0, uchunk1,
               vblock, scores, cnts, semr0, semr1, semv):
    wid = lax.axis_index("s") * NC + lax.axis_index("c")
    iota16 = lax.iota(jnp.int32, L)

    pltpu.sync_copy(items_hbm, ai)

    def zero(t, carry):
        scores[pl.ds(t * L, L)] = jnp.zeros((L,), jnp.float32)
        return carry

    lax.fori_loop(0, (B + L) // L, zero, 0)

    cnts[0] = 0

    def build_outer(p, carry):
        pltpu.sync_copy(users_hbm.at[pl.ds(p * BPW, BPW)], ubuild)

        def build(t, carry2):
            b = p * BPW + t * L + iota16
            u = ubuild[pl.ds(t * L, L)]
            mu = (((u >> 9) & 31) == wid) & (u < UTAIL)
            cu = cnts[0]
            plsc.store_compressed(ulist.at[pl.ds(cu, L)], u, mask=mu)
            plsc.store_compressed(ubl.at[pl.ds(cu, L)], b, mask=mu)
            cnts[0] = cu + plsc.all_reduce_population_count(mu)[0]
            return carry2

        lax.fori_loop(0, BPW // L, build, 0)
        return carry

    lax.fori_loop(0, B // BPW, build_outer, 0)
    ulist[pl.ds(cnts[0], L)] = jnp.full((L,), BIG, jnp.int32)
    cnt = cnts[0]
    nvreg = (cnt + L - 1) >> 4

    def issue(k, chunkbuf, semr):
        cid = wid + NW * k

        @pl.when(cid < NUCH)
        def _():
            off = pl.multiple_of(cid * UCW, UCW)
            pltpu.async_copy(utab_hbm.at[:, pl.ds(off, UCW)], chunkbuf, semr)

    def process(k, chunkbuf, semr):
        cid = wid + NW * k

        @pl.when(cid < NUCH)
        def _():
            pltpu.make_async_copy(
                utab_hbm.at[:, pl.ds(0, UCW)], chunkbuf, semr).wait()

            def scan(t, c2):
                lv = ulist[pl.ds(t * L, L)]
                bv = ubl[pl.ds(t * L, L)]
                m = (lv >> 9) == cid
                plsc.store_compressed(acol.at[pl.ds(c2, L)],
                                      lv & (UCW - 1), mask=m)
                plsc.store_compressed(ab.at[pl.ds(c2, L)], bv, mask=m)
                return c2 + plsc.all_reduce_population_count(m)[0]

            c2 = lax.fori_loop(0, nvreg, scan, 0)
            ngroup = (c2 + L - 1) >> 4

            def group(g, carry2):
                col = acol[pl.ds(g * L, L)] & (UCW - 1)
                bs = ab[pl.ds(g * L, L)] & (B - 1)
                gm = iota16 < (c2 - g * L)
                iv = plsc.load_gather(ai, [bs])
                ivb = iv >> 1
                ivh = (iv & 1) << 6
                pltpu.async_copy(itab_hbm.at[ivb], vblock, semv).wait()
                acc = jnp.zeros((L,), jnp.float32)
                for d in range(D):
                    dd = jnp.full((L,), d, jnp.int32)
                    u = plsc.load_gather(chunkbuf, [dd, col])
                    v = plsc.load_gather(vblock, [iota16, ivh + dd])
                    acc = acc + u * v
                bs_dump = jnp.where(gm, bs, B)
                plsc.store_scatter(scores, [bs_dump], acc)
                return carry2

            lax.fori_loop(0, ngroup, group, 0)

    issue(0, uchunk0, semr0)

    def step(t, carry):
        issue(2 * t + 1, uchunk1, semr1)
        process(2 * t, uchunk0, semr0)

        @pl.when(t + 1 < KU // 2)
        def _():
            issue(2 * t + 2, uchunk0, semr0)

        process(2 * t + 1, uchunk1, semr1)
        return carry

    lax.fori_loop(0, KU // 2, step, 0)

    pltpu.sync_copy(scores.at[pl.ds(0, B)], part_hbm.at[pl.ds(wid * B, B)])


def _reduce_body(users_hbm, items_hbm, part_hbm, utail_hbm, itab_hbm,
                 out_hbm, uidx, iidx, pbuf, utb, vtb, accv, semp, semv):
    wid = lax.axis_index("s") * NC + lax.axis_index("c")
    base = wid * BPW
    iota16 = lax.iota(jnp.int32, L)

    pltpu.sync_copy(users_hbm.at[pl.ds(base, BPW)], uidx)
    pltpu.sync_copy(items_hbm.at[pl.ds(base, BPW)], iidx)
    pltpu.sync_copy(utail_hbm, utb)

    for w2 in range(NW):
        pltpu.async_copy(
            part_hbm.at[pl.ds(w2 * B + base, BPW)], pbuf.at[w2], semp)
    for w2 in range(NW):
        pltpu.make_async_copy(
            part_hbm.at[pl.ds(0, BPW)], pbuf.at[w2], semp).wait()

    def sum_group(g, carry):
        acc = jnp.zeros((L,), jnp.float32)
        for w2 in range(NW):
            acc = acc + pbuf[w2, pl.ds(g * L, L)]
        accv[pl.ds(g * L, L)] = acc
        return carry

    lax.fori_loop(0, BPW // L, sum_group, 0)

    def tail_group(g, carry):
        uvec = uidx[pl.ds(g * L, L)]
        mu = uvec >= UTAIL

        @pl.when(plsc.all_reduce_population_count(mu)[0] > 0)
        def _():
            ivec = iidx[pl.ds(g * L, L)]
            ivb = ivec >> 3
            ivs = ivec & 7
            tuc = jnp.maximum(uvec - UTAIL, 0)
            for j in range(L):
                pltpu.async_copy(itab_hbm.at[ivb[j]], vtb.at[j], semv)
            pltpu.make_async_copy(
                itab_hbm.at[pl.ds(0, L)], vtb, semv).wait()
            acc = jnp.zeros((L,), jnp.float32)
            for d in range(D):
                dd = jnp.full((L,), d, jnp.int32)
                ut = plsc.load_gather(utb, [tuc, dd], mask=mu)
                vt = plsc.load_gather(vtb, [iota16, ivs, dd], mask=mu)
                acc = acc + ut * vt
            old = accv[pl.ds(g * L, L)]
            accv[pl.ds(g * L, L)] = jnp.where(mu, acc, old)

        return carry

    lax.fori_loop(0, BPW // L, tail_group, 0)

    pltpu.sync_copy(accv, out_hbm.at[pl.ds(base, BPW)])


def kernel(users, items, users_table, items_table):
    ut_t = users_table.T
    it3 = items_table.reshape(NI // RB, RB, D)
    utail = users_table[UTAIL:]
    users = users.astype(jnp.int32)
    items = items.astype(jnp.int32)
    mesh = plsc.VectorSubcoreMesh(core_axis_name="c", subcore_axis_name="s")

    main = pl.kernel(
        _main_body,
        out_type=jax.ShapeDtypeStruct((NW * B,), jnp.float32),
        mesh=mesh,
        compiler_params=pltpu.CompilerParams(needs_layout_passes=False),
        scratch_types=[
            pltpu.VMEM((BPW,), jnp.int32),
            pltpu.VMEM((B,), jnp.int32),
            pltpu.VMEM((LCAP + L,), jnp.int32),
            pltpu.VMEM((LCAP + L,), jnp.int32),
            pltpu.VMEM((ACAP + L,), jnp.int32),
            pltpu.VMEM((ACAP + L,), jnp.int32),
            pltpu.VMEM((D, UCW), jnp.float32),
            pltpu.VMEM((D, UCW), jnp.float32),
            pltpu.VMEM((L, 2 * D), jnp.float32),
            pltpu.VMEM((B + L,), jnp.float32),
            pltpu.SMEM((4,), jnp.int32),
            pltpu.SemaphoreType.DMA,
            pltpu.SemaphoreType.DMA,
            pltpu.SemaphoreType.DMA,
        ],
    )
    it2 = items_table.reshape(NI // 2, 2 * D)
    partials = main(users, items, ut_t, it2)

    reduce = pl.kernel(
        _reduce_body,
        out_type=jax.ShapeDtypeStruct((B,), jnp.float32),
        mesh=mesh,
        compiler_params=pltpu.CompilerParams(needs_layout_passes=False),
        scratch_types=[
            pltpu.VMEM((BPW,), jnp.int32),
            pltpu.VMEM((BPW,), jnp.int32),
            pltpu.VMEM((NW, BPW), jnp.float32),
            pltpu.VMEM((D, D), jnp.float32),
            pltpu.VMEM((L, RB, D), jnp.float32),
            pltpu.VMEM((BPW,), jnp.float32),
            pltpu.SemaphoreType.DMA,
            pltpu.SemaphoreType.DMA,
        ],
    )
    return reduce(users, items, partials, utail, it3)
